# Initial kernel scaffold; baseline (speedup 1.0000x reference)
import jax
import jax.numpy as jnp
from jax.experimental import pallas as pl


def kernel(x, params):
    raise NotImplementedError("write your pallas kernel here")


# speedup vs baseline: 1.9990x; 1.9990x over previous
import functools

import numpy as np
import jax
import jax.numpy as jnp
from jax import lax
from jax.experimental import pallas as pl
from jax.experimental.pallas import tpu as pltpu
from jax.experimental.pallas import tpu_sc as plsc

N_POINTS = 262144
N_LEVELS = 12
F = 2
T = 1 << 19
BASE = 16
START_LEVEL = 6
ACTIVE = START_LEVEL
OUT_COLS = N_LEVELS * F

P1 = int(np.array(2654435761, np.uint32).astype(np.int32))
P2 = int(np.array(805459861, np.uint32).astype(np.int32))
HASH_MASK = T - 1

_LEVELS = []
_off = 0
for _l in range(ACTIVE):
    _s = BASE * (2.0 ** _l) - 1.0
    _res = int(np.ceil(_s)) + 1
    _dense = (_res + 1) ** 3
    _hashed = _dense > T
    _LEVELS.append({
        "scale": float(np.float32(_s)),
        "hashed": _hashed,
        "R": _res + 1,
        "off": _off,
    })
    _off += (T if _hashed else _dense)

NC = 2
NS = 16
NW = NC * NS
PW = N_POINTS // NW
C = 256
CHUNKS = PW // C
NGROUPS = C // 16
NIDX = ACTIVE * 8 * C
NROWS = NIDX // 128

@functools.cache
def _build():
    mesh = plsc.VectorSubcoreMesh(
        core_axis_name="c", subcore_axis_name="s",
        num_cores=NC, num_subcores=NS)
    return pl.kernel(
        _grid_encode,
        out_type=jax.ShapeDtypeStruct((N_POINTS * OUT_COLS,), jnp.float32),
        mesh=mesh,
        compiler_params=pltpu.CompilerParams(
            needs_layout_passes=False, use_tc_tiling_on_sc=False),
        scratch_types=[
            pltpu.VMEM((C * 3,), jnp.float32),
            pltpu.VMEM((NROWS, 128), jnp.int32),
            pltpu.VMEM((NIDX,), jnp.float32),
            pltpu.VMEM((NROWS, 128, 2), jnp.float32),
            pltpu.VMEM((C * OUT_COLS,), jnp.float32),
            pltpu.SemaphoreType.DMA,
        ],
    )


def _grid_encode(x_hbm, params_hbm, out_hbm, xv, idxv, wv, rows, outv, sem):
    wid = lax.axis_index("s") * NC + lax.axis_index("c")
    iota = lax.iota(jnp.int32, 16)
    zero16 = jnp.zeros((16,), jnp.float32)
    zi = jnp.zeros((16,), jnp.int32)
    oi = jnp.full((16,), 1, jnp.int32)

    def zero_body(g, carry):
        p24 = (g * 16 + iota) * OUT_COLS
        for col in range(ACTIVE * F, OUT_COLS):
            plsc.store_scatter(outv, [p24 + col], zero16)
        return carry

    lax.fori_loop(0, NGROUPS, zero_body, None)

    def chunk_body(ci, carry):
        base = wid * PW + ci * C
        pltpu.sync_copy(x_hbm.at[pl.ds(base * 3, C * 3)], xv)

        def pass_a(g, acarry):
            p3 = (g * 16 + iota) * 3
            x0 = plsc.load_gather(xv, [p3])
            x1 = plsc.load_gather(xv, [p3 + 1])
            x2 = plsc.load_gather(xv, [p3 + 2])
            for l, lv in enumerate(_LEVELS):
                s = lv["scale"]
                pos0 = x0 * s + 0.5
                pos1 = x1 * s + 0.5
                pos2 = x2 * s + 0.5
                pfi0 = pos0.astype(jnp.int32)
                pfi1 = pos1.astype(jnp.int32)
                pfi2 = pos2.astype(jnp.int32)
                f0 = pos0 - pfi0.astype(jnp.float32)
                f1 = pos1 - pfi1.astype(jnp.float32)
                f2 = pos2 - pfi2.astype(jnp.float32)
                a0 = 1.0 - f0
                a1 = 1.0 - f1
                a2 = 1.0 - f2
                wxy = [a0 * a1, f0 * a1, a0 * f1, f0 * f1]
                wz = [a2, f2]
                t0 = [pfi0, pfi0 + 1]
                if lv["hashed"]:
                    h1 = pfi1 * P1
                    h2 = pfi2 * P2
                    t1 = [h1, h1 + P1]
                    t2 = [h2, h2 + P2]
                else:
                    R = lv["R"]
                    h1 = pfi1 * R
                    h2 = pfi2 * (R * R)
                    t1 = [h1, h1 + R]
                    t2 = [h2, h2 + R * R]
                for j in range(8):
                    b0, b1, b2 = j & 1, (j >> 1) & 1, (j >> 2) & 1
                    if lv["hashed"]:
                        idx = ((t0[b0] ^ t1[b1] ^ t2[b2]) & HASH_MASK) + lv["off"]
                    else:
                        idx = t0[b0] + t1[b1] + t2[b2] + lv["off"]
                    w = wxy[b0 + 2 * b1] * wz[b2]
                    blk = (l * 8 + j) * C
                    flat = blk + g * 16 + iota
                    plsc.store_scatter(
                        idxv, [lax.shift_right_logical(flat, 7), flat & 127], idx)
                    wv[pl.ds(blk + g * 16, 16)] = w
            return acarry

        lax.fori_loop(0, NGROUPS, pass_a, None)

        def fire(i, fcarry):
            pltpu.make_async_copy(params_hbm.at[idxv.at[i]], rows.at[i], sem).start()
            return fcarry

        lax.fori_loop(0, NROWS, fire, None)

        def drain(i, dcarry):
            pltpu.make_async_copy(params_hbm.at[idxv.at[i]], rows.at[i], sem).wait()
            return dcarry

        lax.fori_loop(0, NROWS, drain, None)

        def pass_b(g, bcarry):
            p24 = (g * 16 + iota) * OUT_COLS
            for l in range(ACTIVE):
                acc0 = zero16
                acc1 = zero16
                for j in range(8):
                    blk = (l * 8 + j) * C
                    w = wv[pl.ds(blk + g * 16, 16)]
                    flat = blk + g * 16 + iota
                    r = lax.shift_right_logical(flat, 7)
                    c = flat & 127
                    acc0 = acc0 + w * plsc.load_gather(rows, [r, c, zi])
                    acc1 = acc1 + w * plsc.load_gather(rows, [r, c, oi])
                plsc.store_scatter(outv, [p24 + 2 * l], acc0)
                plsc.store_scatter(outv, [p24 + 2 * l + 1], acc1)
            return bcarry

        lax.fori_loop(0, NGROUPS, pass_b, None)

        pltpu.sync_copy(outv, out_hbm.at[pl.ds(base * OUT_COLS, C * OUT_COLS)])
        return carry

    lax.fori_loop(0, CHUNKS, chunk_body, None)


@jax.jit
def kernel(x, params):
    out = _build()(x.reshape(-1), params)
    return out.reshape(N_POINTS, OUT_COLS)


# speedup vs baseline: 2.0153x; 1.0081x over previous
import functools

import numpy as np
import jax
import jax.numpy as jnp
from jax import lax
from jax.experimental import pallas as pl
from jax.experimental.pallas import tpu as pltpu
from jax.experimental.pallas import tpu_sc as plsc

N_POINTS = 262144
N_LEVELS = 12
F = 2
T = 1 << 19
BASE = 16
START_LEVEL = 6
ACTIVE = START_LEVEL
OUT_COLS = N_LEVELS * F

P1 = int(np.array(2654435761, np.uint32).astype(np.int32))
P2 = int(np.array(805459861, np.uint32).astype(np.int32))
HASH_MASK = T - 1

_LEVELS = []
_off = 0
for _l in range(ACTIVE):
    _s = BASE * (2.0 ** _l) - 1.0
    _res = int(np.ceil(_s)) + 1
    _dense = (_res + 1) ** 3
    _hashed = _dense > T
    _LEVELS.append({
        "scale": float(np.float32(_s)),
        "hashed": _hashed,
        "R": _res + 1,
        "off": _off,
    })
    _off += (T if _hashed else _dense)

NC = 2
NS = 16
NW = NC * NS
PW = N_POINTS // NW
C = 128
CHUNKS = PW // C
NGROUPS = C // 16
L0_ROWS = 4913
NIDX = (ACTIVE - 1) * 8 * C
NROWS = NIDX // 128


@functools.cache
def _build():
    mesh = plsc.VectorSubcoreMesh(
        core_axis_name="c", subcore_axis_name="s",
        num_cores=NC, num_subcores=NS)
    return pl.kernel(
        _grid_encode,
        out_type=jax.ShapeDtypeStruct((N_POINTS * OUT_COLS,), jnp.float32),
        mesh=mesh,
        compiler_params=pltpu.CompilerParams(
            needs_layout_passes=False, use_tc_tiling_on_sc=False),
        scratch_types=[
            pltpu.VMEM((C * 3,), jnp.float32),
            pltpu.VMEM((L0_ROWS, 2), jnp.float32),
            pltpu.VMEM((NROWS, 128), jnp.int32),
            pltpu.VMEM((NIDX,), jnp.float32),
            pltpu.VMEM((NROWS, 128, 2), jnp.float32),
            pltpu.VMEM((C * OUT_COLS,), jnp.float32),
            pltpu.SemaphoreType.DMA,
        ],
    )


def _grid_encode(x_hbm, params_hbm, out_hbm,
                 xv, tbl0, idxv, wv, rows, outv, sem):
    wid = lax.axis_index("s") * NC + lax.axis_index("c")
    iota = lax.iota(jnp.int32, 16)
    zero16 = jnp.zeros((16,), jnp.float32)
    zi = jnp.zeros((16,), jnp.int32)
    oi = jnp.full((16,), 1, jnp.int32)

    def zero_body(g, carry):
        p24 = (g * 16 + iota) * OUT_COLS
        for col in range(ACTIVE * F, OUT_COLS):
            plsc.store_scatter(outv, [p24 + col], zero16)
        return carry

    lax.fori_loop(0, NGROUPS, zero_body, None)

    pltpu.sync_copy(params_hbm.at[pl.ds(0, L0_ROWS)], tbl0)

    def chunk_body(ci, carry):
        base = wid * PW + ci * C
        pltpu.sync_copy(x_hbm.at[pl.ds(base * 3, C * 3)], xv)

        def pass_a(g, acarry):
            p3 = (g * 16 + iota) * 3
            p24 = (g * 16 + iota) * OUT_COLS
            x0 = plsc.load_gather(xv, [p3])
            x1 = plsc.load_gather(xv, [p3 + 1])
            x2 = plsc.load_gather(xv, [p3 + 2])
            for l, lv in enumerate(_LEVELS):
                s = lv["scale"]
                pos0 = x0 * s + 0.5
                pos1 = x1 * s + 0.5
                pos2 = x2 * s + 0.5
                pfi0 = pos0.astype(jnp.int32)
                pfi1 = pos1.astype(jnp.int32)
                pfi2 = pos2.astype(jnp.int32)
                f0 = pos0 - pfi0.astype(jnp.float32)
                f1 = pos1 - pfi1.astype(jnp.float32)
                f2 = pos2 - pfi2.astype(jnp.float32)
                a0 = 1.0 - f0
                a1 = 1.0 - f1
                a2 = 1.0 - f2
                wxy = [a0 * a1, f0 * a1, a0 * f1, f0 * f1]
                wz = [a2, f2]
                t0 = [pfi0, pfi0 + 1]
                if lv["hashed"]:
                    h1 = pfi1 * P1
                    h2 = pfi2 * P2
                    t1 = [h1, h1 + P1]
                    t2 = [h2, h2 + P2]
                else:
                    R = lv["R"]
                    h1 = pfi1 * R
                    h2 = pfi2 * (R * R)
                    t1 = [h1, h1 + R]
                    t2 = [h2, h2 + R * R]
                if l == 0:
                    acc0 = zero16
                    acc1 = zero16
                    for j in range(8):
                        b0, b1, b2 = j & 1, (j >> 1) & 1, (j >> 2) & 1
                        idx = t0[b0] + t1[b1] + t2[b2]
                        w = wxy[b0 + 2 * b1] * wz[b2]
                        acc0 = acc0 + w * plsc.load_gather(tbl0, [idx, zi])
                        acc1 = acc1 + w * plsc.load_gather(tbl0, [idx, oi])
                    plsc.store_scatter(outv, [p24], acc0)
                    plsc.store_scatter(outv, [p24 + 1], acc1)
                    continue
                for j in range(8):
                    b0, b1, b2 = j & 1, (j >> 1) & 1, (j >> 2) & 1
                    if lv["hashed"]:
                        idx = ((t0[b0] ^ t1[b1] ^ t2[b2]) & HASH_MASK) + lv["off"]
                    else:
                        idx = t0[b0] + t1[b1] + t2[b2] + lv["off"]
                    w = wxy[b0 + 2 * b1] * wz[b2]
                    blk = ((l - 1) * 8 + j) * C
                    flat = blk + g * 16 + iota
                    plsc.store_scatter(
                        idxv, [lax.shift_right_logical(flat, 7), flat & 127], idx)
                    wv[pl.ds(blk + g * 16, 16)] = w
            return acarry

        lax.fori_loop(0, NGROUPS, pass_a, None)

        def fire(i, fcarry):
            pltpu.make_async_copy(params_hbm.at[idxv.at[i]], rows.at[i], sem).start()
            return fcarry

        lax.fori_loop(0, NROWS, fire, None)

        def drain(i, dcarry):
            pltpu.make_async_copy(params_hbm.at[idxv.at[i]], rows.at[i], sem).wait()
            return dcarry

        lax.fori_loop(0, NROWS, drain, None)

        def pass_b(g, bcarry):
            p24 = (g * 16 + iota) * OUT_COLS
            for l in range(1, ACTIVE):
                acc0 = zero16
                acc1 = zero16
                for j in range(8):
                    blk = ((l - 1) * 8 + j) * C
                    w = wv[pl.ds(blk + g * 16, 16)]
                    flat = blk + g * 16 + iota
                    r = lax.shift_right_logical(flat, 7)
                    c = flat & 127
                    acc0 = acc0 + w * plsc.load_gather(rows, [r, c, zi])
                    acc1 = acc1 + w * plsc.load_gather(rows, [r, c, oi])
                plsc.store_scatter(outv, [p24 + 2 * l], acc0)
                plsc.store_scatter(outv, [p24 + 2 * l + 1], acc1)
            return bcarry

        lax.fori_loop(0, NGROUPS, pass_b, None)

        pltpu.sync_copy(outv, out_hbm.at[pl.ds(base * OUT_COLS, C * OUT_COLS)])
        return carry

    lax.fori_loop(0, CHUNKS, chunk_body, None)


@jax.jit
def kernel(x, params):
    out = _build()(x.reshape(-1), params)
    return out.reshape(N_POINTS, OUT_COLS)


# speedup vs baseline: 2.3149x; 1.1487x over previous
import functools

import numpy as np
import jax
import jax.numpy as jnp
from jax import lax
from jax.experimental import pallas as pl
from jax.experimental.pallas import tpu as pltpu
from jax.experimental.pallas import tpu_sc as plsc

N_POINTS = 262144
N_LEVELS = 12
F = 2
T = 1 << 19
BASE = 16
START_LEVEL = 6
ACTIVE = START_LEVEL
OUT_COLS = N_LEVELS * F

P1 = int(np.array(2654435761, np.uint32).astype(np.int32))
P2 = int(np.array(805459861, np.uint32).astype(np.int32))
HASH_MASK = T - 1

_LEVELS = []
_off = 0
for _l in range(ACTIVE):
    _s = BASE * (2.0 ** _l) - 1.0
    _res = int(np.ceil(_s)) + 1
    _dense = (_res + 1) ** 3
    _hashed = _dense > T
    _LEVELS.append({
        "scale": float(np.float32(_s)),
        "hashed": _hashed,
        "R": _res + 1,
        "off": _off,
    })
    _off += (T if _hashed else _dense)

NC = 2
NS = 16
NW = NC * NS
PW = N_POINTS // NW
C = 128
CHUNKS = PW // C
NGROUPS = C // 16
L0_ROWS = 4913
L0_PAD = 9832
NIDX = (ACTIVE - 1) * 8 * C
NROWS = NIDX // 128


@functools.cache
def _build():
    mesh = plsc.VectorSubcoreMesh(
        core_axis_name="c", subcore_axis_name="s",
        num_cores=NC, num_subcores=NS)
    return pl.kernel(
        _grid_encode,
        out_type=jax.ShapeDtypeStruct((N_POINTS * OUT_COLS,), jnp.float32),
        mesh=mesh,
        compiler_params=pltpu.CompilerParams(
            needs_layout_passes=False, use_tc_tiling_on_sc=False),
        scratch_types=[
            pltpu.VMEM((C * 3,), jnp.float32),
            pltpu.VMEM((L0_PAD,), jnp.float32),
            pltpu.VMEM((NROWS, 128), jnp.int32),
            pltpu.VMEM((NROWS, 128), jnp.int32),
            pltpu.VMEM((NIDX,), jnp.float32),
            pltpu.VMEM((NROWS, 128), jnp.float32),
            pltpu.VMEM((NROWS, 128), jnp.float32),
            pltpu.VMEM((C * OUT_COLS,), jnp.float32),
            pltpu.SemaphoreType.DMA,
        ],
    )


def _grid_encode(x_hbm, pflat_hbm, out_hbm,
                 xv, tbl0, idxv0, idxv1, wv, rows0, rows1, outv, sem):
    wid = lax.axis_index("s") * NC + lax.axis_index("c")
    iota = lax.iota(jnp.int32, 16)
    zero16 = jnp.zeros((16,), jnp.float32)
    zi = jnp.zeros((16,), jnp.int32)
    oi = jnp.full((16,), 1, jnp.int32)

    def zero_body(g, carry):
        p24 = (g * 16 + iota) * OUT_COLS
        for col in range(ACTIVE * F, OUT_COLS):
            plsc.store_scatter(outv, [p24 + col], zero16)
        return carry

    lax.fori_loop(0, NGROUPS, zero_body, None)

    pltpu.sync_copy(pflat_hbm.at[pl.ds(0, L0_PAD)], tbl0)

    def chunk_body(ci, carry):
        base = wid * PW + ci * C
        pltpu.sync_copy(x_hbm.at[pl.ds(base * 3, C * 3)], xv)

        def pass_a(g, acarry):
            p3 = (g * 16 + iota) * 3
            p24 = (g * 16 + iota) * OUT_COLS
            x0 = plsc.load_gather(xv, [p3])
            x1 = plsc.load_gather(xv, [p3 + 1])
            x2 = plsc.load_gather(xv, [p3 + 2])
            for l, lv in enumerate(_LEVELS):
                s = lv["scale"]
                pos0 = x0 * s + 0.5
                pos1 = x1 * s + 0.5
                pos2 = x2 * s + 0.5
                pfi0 = pos0.astype(jnp.int32)
                pfi1 = pos1.astype(jnp.int32)
                pfi2 = pos2.astype(jnp.int32)
                f0 = pos0 - pfi0.astype(jnp.float32)
                f1 = pos1 - pfi1.astype(jnp.float32)
                f2 = pos2 - pfi2.astype(jnp.float32)
                a0 = 1.0 - f0
                a1 = 1.0 - f1
                a2 = 1.0 - f2
                wxy = [a0 * a1, f0 * a1, a0 * f1, f0 * f1]
                wz = [a2, f2]
                t0 = [pfi0, pfi0 + 1]
                if lv["hashed"]:
                    h1 = pfi1 * P1
                    h2 = pfi2 * P2
                    t1 = [h1, h1 + P1]
                    t2 = [h2, h2 + P2]
                else:
                    R = lv["R"]
                    h1 = pfi1 * R
                    h2 = pfi2 * (R * R)
                    t1 = [h1, h1 + R]
                    t2 = [h2, h2 + R * R]
                if l == 0:
                    acc0 = zero16
                    acc1 = zero16
                    for j in range(8):
                        b0, b1, b2 = j & 1, (j >> 1) & 1, (j >> 2) & 1
                        idx2 = (t0[b0] + t1[b1] + t2[b2]) * 2
                        w = wxy[b0 + 2 * b1] * wz[b2]
                        acc0 = acc0 + w * plsc.load_gather(tbl0, [idx2])
                        acc1 = acc1 + w * plsc.load_gather(tbl0, [idx2 + 1])
                    plsc.store_scatter(outv, [p24], acc0)
                    plsc.store_scatter(outv, [p24 + 1], acc1)
                    continue
                for j in range(8):
                    b0, b1, b2 = j & 1, (j >> 1) & 1, (j >> 2) & 1
                    if lv["hashed"]:
                        idx = ((t0[b0] ^ t1[b1] ^ t2[b2]) & HASH_MASK) + lv["off"]
                    else:
                        idx = t0[b0] + t1[b1] + t2[b2] + lv["off"]
                    w = wxy[b0 + 2 * b1] * wz[b2]
                    idx2 = idx * 2
                    blk = ((l - 1) * 8 + j) * C
                    flat = blk + g * 16 + iota
                    r = lax.shift_right_logical(flat, 7)
                    cc = flat & 127
                    plsc.store_scatter(idxv0, [r, cc], idx2)
                    plsc.store_scatter(idxv1, [r, cc], idx2 + 1)
                    wv[pl.ds(blk + g * 16, 16)] = w
            return acarry

        lax.fori_loop(0, NGROUPS, pass_a, None)

        def fire(i, fcarry):
            pltpu.make_async_copy(
                pflat_hbm.at[idxv0.at[i]], rows0.at[i], sem).start()
            pltpu.make_async_copy(
                pflat_hbm.at[idxv1.at[i]], rows1.at[i], sem).start()
            return fcarry

        lax.fori_loop(0, NROWS, fire, None)

        def drain(i, dcarry):
            pltpu.make_async_copy(
                pflat_hbm.at[idxv0.at[i]], rows0.at[i], sem).wait()
            pltpu.make_async_copy(
                pflat_hbm.at[idxv1.at[i]], rows1.at[i], sem).wait()
            return dcarry

        lax.fori_loop(0, NROWS, drain, None)

        def pass_b(g, bcarry):
            p24 = (g * 16 + iota) * OUT_COLS
            for l in range(1, ACTIVE):
                acc0 = zero16
                acc1 = zero16
                for j in range(8):
                    blk = ((l - 1) * 8 + j) * C
                    w = wv[pl.ds(blk + g * 16, 16)]
                    flat = blk + g * 16 + iota
                    r = lax.shift_right_logical(flat, 7)
                    c = flat & 127
                    acc0 = acc0 + w * plsc.load_gather(rows0, [r, c])
                    acc1 = acc1 + w * plsc.load_gather(rows1, [r, c])
                plsc.store_scatter(outv, [p24 + 2 * l], acc0)
                plsc.store_scatter(outv, [p24 + 2 * l + 1], acc1)
            return bcarry

        lax.fori_loop(0, NGROUPS, pass_b, None)

        pltpu.sync_copy(outv, out_hbm.at[pl.ds(base * OUT_COLS, C * OUT_COLS)])
        return carry

    lax.fori_loop(0, CHUNKS, chunk_body, None)


@jax.jit
def kernel(x, params):
    out = _build()(x.reshape(-1), params.reshape(-1))
    return out.reshape(N_POINTS, OUT_COLS)
